# speedup vs baseline: 1.0012x; 1.0012x over previous
import functools

import jax
import jax.numpy as jnp
from jax import lax
from jax.experimental import pallas as pl
from jax.experimental.pallas import tpu as pltpu
from jax.experimental.pallas import tpu_sc as plsc

_B = 4
_T = 4096
_D = 768
_N = _B * _T
_NC = 2
_NS = 16
_NW = _NC * _NS
_PER_W = _N // _NW
_CHUNK = 64
_NCHUNK = _PER_W // _CHUNK
_LANES = 16
_GRP = _D // _LANES


def _make_emb_kernel():
    mesh = plsc.VectorSubcoreMesh(core_axis_name="c", subcore_axis_name="s")

    @functools.partial(
        pl.kernel,
        out_type=jax.ShapeDtypeStruct((_N, _D), jnp.float32),
        mesh=mesh,
        scratch_types=[
            pltpu.VMEM((_CHUNK,), jnp.int32),
            pltpu.VMEM((_CHUNK, _D), jnp.float32),
            pltpu.VMEM((_CHUNK, _D), jnp.float32),
            pltpu.SemaphoreType.DMA,
        ],
    )
    def emb(x_hbm, tok_hbm, pos_hbm, out_hbm, idx_v, tok_v, pos_v, sem):
        wid = lax.axis_index("s") * _NC + lax.axis_index("c")
        base = wid * _PER_W

        def chunk_body(ci, _):
            r0 = base + ci * _CHUNK
            p0 = lax.rem(r0, _T)
            pltpu.sync_copy(x_hbm.at[pl.ds(r0, _CHUNK)], idx_v)
            gather = pltpu.async_copy(tok_hbm.at[idx_v], tok_v, sem)
            pltpu.sync_copy(pos_hbm.at[pl.ds(p0, _CHUNK)], pos_v)
            gather.wait()

            def row_body(i, _):
                for j in range(_GRP):
                    s = pl.ds(j * _LANES, _LANES)
                    tok_v[i, s] = tok_v[i, s] + pos_v[i, s]
                return 0

            lax.fori_loop(0, _CHUNK, row_body, 0)
            pltpu.sync_copy(tok_v, out_hbm.at[pl.ds(r0, _CHUNK)])
            return 0

        lax.fori_loop(0, _NCHUNK, chunk_body, 0)

    return emb


_emb = _make_emb_kernel()


def kernel(x, token_table, pos_table):
    xf = x.reshape(_N).astype(jnp.int32)
    out = _emb(xf, token_table, pos_table)
    return out.reshape(_B, _T, _D)
